# baseline (speedup 1.0000x reference)
import functools

import jax
import jax.numpy as jnp
from jax import lax
from jax.experimental import pallas as pl
from jax.experimental.pallas import tpu as pltpu
from jax.experimental.pallas import tpu_sc as plsc

BS, SEQ, IN_DIM, EMB = 4, 4096, 512, 1024
HID = EMB // 2
KK = max(1, int((SEQ - 2) * 0.3))
KPAD = 1280
PT = KPAD // 8
ROWS = BS * KPAD
NCH = SEQ // 16


def _sc_body(text_hbm, feat_hbm, out_hbm, lens_hbm,
             trow, idxa, idxb, rowsa, rowsb, nzv, sema, semb):
    c = lax.axis_index("c")
    sub = lax.axis_index("s")
    wid = 16 * c + sub
    b = wid // 8
    s = wid % 8
    lo = s * PT
    dstbase = b * KPAD + lo

    pltpu.sync_copy(text_hbm.at[pl.ds(b * SEQ, SEQ)], trow)

    lane = lax.iota(jnp.int32, 16)

    def abody(i, zt):
        t = trow[pl.ds(i * 16, 16)]
        return zt + plsc.all_reduce_population_count(t == 0)
    z_total = lax.fori_loop(0, NCH, abody, jnp.zeros((16,), jnp.int32))

    @pl.when(s == 0)
    def _():
        nzv[...] = z_total
        pltpu.sync_copy(nzv, lens_hbm.at[b])

    lo_vec = jnp.full((16,), lo, jnp.int32)
    src_base = jnp.full((16,), b * SEQ, jnp.int32) + lane

    zt0 = z_total[0]

    def bbody(i, z_run):
        t = trow[pl.ds(i * 16, 16)]
        zb = t == 0
        zcnt = plsc.all_reduce_population_count(zb)

        @pl.when((zcnt[0] > 0)
                 | ((i * 16 + 15 >= lo - zt0) & (i * 16 < lo + PT)))
        def _():
            zc = plsc.cumsum(zb.astype(jnp.int32))
            pos = i * 16 + lane
            rank = jnp.where(zb, z_run + zc,
                             z_total + pos + 1 - z_run - zc) - 1
            slot = rank - lo_vec
            ma = (slot >= 0) & (slot < 128)
            mb = (slot >= 128) & (slot < PT)
            srcv = src_base + i * 16
            plsc.store_scatter(idxa, [jnp.clip(slot, 0, 127)], srcv, mask=ma)
            plsc.store_scatter(idxb, [jnp.clip(slot - 128, 0, PT - 129)],
                               srcv, mask=mb)
        return z_run + zcnt
    lax.fori_loop(0, NCH, bbody, jnp.zeros((16,), jnp.int32))

    ca = pltpu.async_copy(feat_hbm.at[idxa], rowsa, sema)
    cb = pltpu.async_copy(feat_hbm.at[idxb], rowsb, semb)
    ca.wait()
    pltpu.sync_copy(rowsa, out_hbm.at[pl.ds(dstbase, 128)])
    cb.wait()
    pltpu.sync_copy(rowsb, out_hbm.at[pl.ds(dstbase + 128, PT - 128)])


@functools.partial(
    pl.kernel,
    mesh=plsc.VectorSubcoreMesh(core_axis_name="c", subcore_axis_name="s"),
    compiler_params=pltpu.CompilerParams(needs_layout_passes=False),
    out_type=[
        jax.ShapeDtypeStruct((ROWS, IN_DIM), jnp.float32),
        jax.ShapeDtypeStruct((BS, 16), jnp.int32),
    ],
    scratch_types=[
        pltpu.VMEM((SEQ,), jnp.int32),
        pltpu.VMEM((128,), jnp.int32),
        pltpu.VMEM((PT - 128,), jnp.int32),
        pltpu.VMEM((128, IN_DIM), jnp.float32),
        pltpu.VMEM((PT - 128, IN_DIM), jnp.float32),
        pltpu.VMEM((16,), jnp.int32),
        pltpu.SemaphoreType.DMA,
        pltpu.SemaphoreType.DMA,
    ],
)
def _sc_gather(text_hbm, feat_hbm, out_hbm, lens_hbm, *scratch):
    _sc_body(text_hbm, feat_hbm, out_hbm, lens_hbm, *scratch)


def _tc_body(x_ref, wlt_ref, w0t_ref, w1t_ref, blin_ref, b0_ref, b1_ref,
             g0_ref, be0_ref, lens_ref, out_ref):
    ridx = lax.broadcasted_iota(jnp.int32, (ROWS, 1), 0)
    row_ok = ridx % KPAD < KK

    x = jnp.where(row_ok, x_ref[...], 0.0)
    ssq = jnp.sum(x * x, axis=1, keepdims=True)
    xn = x / jnp.maximum(jnp.sqrt(ssq), 1e-6)
    xb = xn.astype(jnp.bfloat16)

    h = jnp.dot(xb, w0t_ref[...].astype(jnp.bfloat16),
                preferred_element_type=jnp.float32) + b0_ref[...]
    denom = jnp.float32(BS * KK)
    mu = jnp.sum(jnp.where(row_ok, h, 0.0), axis=0, keepdims=True) / denom
    d = h - mu
    var = jnp.sum(jnp.where(row_ok, d * d, 0.0), axis=0, keepdims=True) / denom
    hn = d / jnp.sqrt(var + 1e-5) * g0_ref[...] + be0_ref[...]
    r = jnp.maximum(hn, 0.0)

    fused = (jnp.dot(r.astype(jnp.bfloat16), w1t_ref[...].astype(jnp.bfloat16),
                     preferred_element_type=jnp.float32)
             + jnp.dot(xb, wlt_ref[...].astype(jnp.bfloat16),
                       preferred_element_type=jnp.float32)
             + b1_ref[...] + blin_ref[...])

    neg = jnp.float32(-jnp.inf)
    for b in range(BS):
        plen = jnp.clip(SEQ - lens_ref[b, 0] - 2, 1, KK)
        seg = fused[b * KPAD:(b + 1) * KPAD, :]
        pm = lax.broadcasted_iota(jnp.int32, (KPAD, 1), 0) < plen
        out_ref[b, :] = jnp.max(jnp.where(pm, seg, neg), axis=0)


def _tc_dense(x, wlt, w0t, w1t, blin, b0, b1, g0, be0, lens):
    vspec = pl.BlockSpec(memory_space=pltpu.VMEM)
    return pl.pallas_call(
        _tc_body,
        out_shape=jax.ShapeDtypeStruct((BS, EMB), jnp.float32),
        in_specs=[vspec] * 9 + [pl.BlockSpec(memory_space=pltpu.SMEM)],
        out_specs=vspec,
        compiler_params=pltpu.CompilerParams(
            vmem_limit_bytes=128 * 1024 * 1024),
    )(x, wlt, w0t, w1t, blin, b0, b1, g0, be0, lens)


def kernel(features, text, atten, W_lin, b_lin, W0, b0, g0, be0, W1, b1):
    del atten
    feat_flat = features.reshape(BS * SEQ, IN_DIM)
    text_flat = text.reshape(BS * SEQ).astype(jnp.int32)

    feats_sc, lens = _sc_gather(text_flat, feat_flat)

    out = _tc_dense(
        feats_sc,
        W_lin.T, W0.T, W1.T,
        b_lin.reshape(1, EMB), b0.reshape(1, HID), b1.reshape(1, EMB),
        g0.reshape(1, HID), be0.reshape(1, HID),
        lens,
    )
    return out.astype(jnp.float32)


# speedup vs baseline: 1.1052x; 1.1052x over previous
import functools

import jax
import jax.numpy as jnp
from jax import lax
from jax.experimental import pallas as pl
from jax.experimental.pallas import tpu as pltpu
from jax.experimental.pallas import tpu_sc as plsc

BS, SEQ, IN_DIM, EMB = 4, 4096, 512, 1024
HID = EMB // 2
KK = max(1, int((SEQ - 2) * 0.3))
KPAD = 1280
PT = KPAD // 8
ROWS = BS * KPAD
NCH = SEQ // 16


def _sc_body(text_hbm, feat_hbm, out_hbm, lens_hbm,
             trow, idxa, idxb, rowsa, rowsb, nzv, sema, semb):
    c = lax.axis_index("c")
    sub = lax.axis_index("s")
    wid = 16 * c + sub
    b = wid // 8
    s = wid % 8
    lo = s * PT
    dstbase = b * KPAD + lo

    pltpu.sync_copy(text_hbm.at[pl.ds(b * SEQ, SEQ)], trow)

    lane = lax.iota(jnp.int32, 16)

    def abody(i, zt):
        t = trow[pl.ds(i * 16, 16)]
        return zt + plsc.all_reduce_population_count(t == 0)
    z_total = lax.fori_loop(0, NCH, abody, jnp.zeros((16,), jnp.int32))

    @pl.when(s == 0)
    def _():
        nzv[...] = z_total
        pltpu.sync_copy(nzv, lens_hbm.at[b])

    lo_vec = jnp.full((16,), lo, jnp.int32)
    src_base = jnp.full((16,), b * SEQ, jnp.int32) + lane

    def bbody(i, z_run):
        t = trow[pl.ds(i * 16, 16)]
        zb = t == 0
        zc = plsc.cumsum(zb.astype(jnp.int32))
        pos = i * 16 + lane
        rank = jnp.where(zb, z_run + zc,
                         z_total + pos + 1 - z_run - zc) - 1
        slot = rank - lo_vec
        ma = (slot >= 0) & (slot < 128)
        mb = (slot >= 128) & (slot < PT)
        srcv = src_base + i * 16
        plsc.store_scatter(idxa, [jnp.clip(slot, 0, 127)], srcv, mask=ma)
        plsc.store_scatter(idxb, [jnp.clip(slot - 128, 0, PT - 129)],
                           srcv, mask=mb)
        return z_run + plsc.all_reduce_population_count(zb)
    lax.fori_loop(0, NCH, bbody, jnp.zeros((16,), jnp.int32))

    ca = pltpu.async_copy(feat_hbm.at[idxa], rowsa, sema)
    cb = pltpu.async_copy(feat_hbm.at[idxb], rowsb, semb)
    ca.wait()
    pltpu.sync_copy(rowsa, out_hbm.at[pl.ds(dstbase, 128)])
    cb.wait()
    pltpu.sync_copy(rowsb, out_hbm.at[pl.ds(dstbase + 128, PT - 128)])


@functools.partial(
    pl.kernel,
    mesh=plsc.VectorSubcoreMesh(core_axis_name="c", subcore_axis_name="s"),
    compiler_params=pltpu.CompilerParams(needs_layout_passes=False),
    out_type=[
        jax.ShapeDtypeStruct((ROWS, IN_DIM), jnp.float32),
        jax.ShapeDtypeStruct((BS, 16), jnp.int32),
    ],
    scratch_types=[
        pltpu.VMEM((SEQ,), jnp.int32),
        pltpu.VMEM((128,), jnp.int32),
        pltpu.VMEM((PT - 128,), jnp.int32),
        pltpu.VMEM((128, IN_DIM), jnp.float32),
        pltpu.VMEM((PT - 128, IN_DIM), jnp.float32),
        pltpu.VMEM((16,), jnp.int32),
        pltpu.SemaphoreType.DMA,
        pltpu.SemaphoreType.DMA,
    ],
)
def _sc_gather(text_hbm, feat_hbm, out_hbm, lens_hbm, *scratch):
    _sc_body(text_hbm, feat_hbm, out_hbm, lens_hbm, *scratch)


NBLK = 8
BLK = ROWS // NBLK


def _tc_body(x_hbm, wlt_ref, w0t_ref, w1t_ref, blin_ref, b0_ref, b1_ref,
             g0_ref, be0_ref, lens_ref, out_ref, xbuf, hbuf, sems):
    copies = []
    for k in range(NBLK):
        copies.append(pltpu.make_async_copy(
            x_hbm.at[pl.ds(k * BLK, BLK)], xbuf.at[pl.ds(k * BLK, BLK)],
            sems.at[k]))
        copies[-1].start()

    denom = jnp.float32(BS * KK)
    s1 = jnp.zeros((1, HID), jnp.float32)
    s2 = jnp.zeros((1, HID), jnp.float32)
    half_ok = lax.broadcasted_iota(jnp.int32, (BLK, 1), 0) < (KK - BLK)
    for k in range(NBLK):
        copies[k].wait()
        xk = xbuf[k * BLK:(k + 1) * BLK, :]
        if k % 2 == 1:
            xk = jnp.where(half_ok, xk, 0.0)
        ssq = jnp.sum(xk * xk, axis=1, keepdims=True)
        xn = xk / jnp.maximum(jnp.sqrt(ssq), 1e-6)
        xbuf[k * BLK:(k + 1) * BLK, :] = xn
        h = jnp.dot(xn, w0t_ref[...],
                    preferred_element_type=jnp.float32) + b0_ref[...]
        hm = jnp.where(half_ok, h, 0.0) if k % 2 == 1 else h
        s1 = s1 + jnp.sum(hm, axis=0, keepdims=True)
        s2 = s2 + jnp.sum(hm * hm, axis=0, keepdims=True)
        hbuf[k * BLK:(k + 1) * BLK, :] = h

    mu = s1 / denom
    var = s2 / denom - mu * mu
    rstd = 1.0 / jnp.sqrt(var + 1e-5)

    neg = jnp.float32(-jnp.inf)
    biota = lax.broadcasted_iota(jnp.int32, (BLK, 1), 0)
    for k in range(NBLK):
        b = k // 2
        h = hbuf[k * BLK:(k + 1) * BLK, :]
        hn = (h - mu) * rstd * g0_ref[...] + be0_ref[...]
        r = jnp.maximum(hn, 0.0)
        fused = (jnp.dot(r, w1t_ref[...], preferred_element_type=jnp.float32)
                 + jnp.dot(xbuf[k * BLK:(k + 1) * BLK, :], wlt_ref[...],
                           preferred_element_type=jnp.float32)
                 + b1_ref[...] + blin_ref[...])
        plen = jnp.clip(SEQ - lens_ref[b, 0] - 2, 1, KK)
        pm = biota + (k % 2) * BLK < plen
        bmax = jnp.max(jnp.where(pm, fused, neg), axis=0)
        if k % 2 == 0:
            out_ref[b, :] = bmax
        else:
            out_ref[b, :] = jnp.maximum(out_ref[b, :], bmax)


def _tc_dense(x, wlt, w0t, w1t, blin, b0, b1, g0, be0, lens):
    vspec = pl.BlockSpec(memory_space=pltpu.VMEM)
    return pl.pallas_call(
        _tc_body,
        out_shape=jax.ShapeDtypeStruct((BS, EMB), jnp.float32),
        in_specs=[pl.BlockSpec(memory_space=pl.ANY)] + [vspec] * 8
        + [pl.BlockSpec(memory_space=pltpu.SMEM)],
        out_specs=vspec,
        scratch_shapes=[
            pltpu.VMEM((ROWS, IN_DIM), jnp.float32),
            pltpu.VMEM((ROWS, HID), jnp.float32),
            pltpu.SemaphoreType.DMA((NBLK,)),
        ],
        compiler_params=pltpu.CompilerParams(
            vmem_limit_bytes=128 * 1024 * 1024),
    )(x, wlt, w0t, w1t, blin, b0, b1, g0, be0, lens)


def kernel(features, text, atten, W_lin, b_lin, W0, b0, g0, be0, W1, b1):
    del atten
    feat_flat = features.reshape(BS * SEQ, IN_DIM)
    text_flat = text.reshape(BS * SEQ).astype(jnp.int32)

    feats_sc, lens = _sc_gather(text_flat, feat_flat)

    out = _tc_dense(
        feats_sc,
        W_lin.T, W0.T, W1.T,
        b_lin.reshape(1, EMB), b0.reshape(1, HID), b1.reshape(1, EMB),
        g0.reshape(1, HID), be0.reshape(1, HID),
        lens,
    )
    return out.astype(jnp.float32)


# speedup vs baseline: 1.1588x; 1.0485x over previous
import functools

import jax
import jax.numpy as jnp
from jax import lax
from jax.experimental import pallas as pl
from jax.experimental.pallas import tpu as pltpu
from jax.experimental.pallas import tpu_sc as plsc

BS, SEQ, IN_DIM, EMB = 4, 4096, 512, 1024
HID = EMB // 2
KK = max(1, int((SEQ - 2) * 0.3))
KPAD = 1280
PT = KPAD // 8
ROWS = BS * KPAD
NCH = SEQ // 16


def _sc_body(text_hbm, feat_hbm, out_hbm, lens_hbm,
             trow, idxa, idxb, rowsa, rowsb, nzv, sema, semb):
    c = lax.axis_index("c")
    sub = lax.axis_index("s")
    wid = 16 * c + sub
    b = wid // 8
    s = wid % 8
    lo = s * PT
    dstbase = b * KPAD + lo

    pltpu.sync_copy(text_hbm.at[pl.ds(b * SEQ, SEQ)], trow)

    lane = lax.iota(jnp.int32, 16)

    def abody(i, zt):
        t = trow[pl.ds(i * 16, 16)]
        return zt + plsc.all_reduce_population_count(t == 0)
    z_total = lax.fori_loop(0, NCH, abody, jnp.zeros((16,), jnp.int32))

    @pl.when(s == 0)
    def _():
        nzv[...] = z_total
        pltpu.sync_copy(nzv, lens_hbm.at[b])

    lo_vec = jnp.full((16,), lo, jnp.int32)
    src_base = jnp.full((16,), b * SEQ, jnp.int32) + lane

    def bbody(i, z_run):
        t = trow[pl.ds(i * 16, 16)]
        zb = t == 0
        zc = plsc.cumsum(zb.astype(jnp.int32))
        pos = i * 16 + lane
        rank = jnp.where(zb, z_run + zc,
                         z_total + pos + 1 - z_run - zc) - 1
        slot = rank - lo_vec
        ma = (slot >= 0) & (slot < 128)
        mb = (slot >= 128) & (slot < PT)
        srcv = src_base + i * 16
        plsc.store_scatter(idxa, [jnp.clip(slot, 0, 127)], srcv, mask=ma)
        plsc.store_scatter(idxb, [jnp.clip(slot - 128, 0, PT - 129)],
                           srcv, mask=mb)
        return z_run + plsc.all_reduce_population_count(zb)

    zt0 = z_total[0]
    chunk_end = jnp.minimum(NCH, (lo + PT + zt0 + 15) // 16)
    z_run1 = lax.fori_loop(0, chunk_end, bbody, jnp.zeros((16,), jnp.int32))

    @pl.when(zt0 > lo)
    def _():
        lax.fori_loop(chunk_end, NCH, bbody, z_run1)

    ca = pltpu.async_copy(feat_hbm.at[idxa], rowsa, sema)
    cb = pltpu.async_copy(feat_hbm.at[idxb], rowsb, semb)
    ca.wait()
    pltpu.sync_copy(rowsa, out_hbm.at[pl.ds(dstbase, 128)])
    cb.wait()
    pltpu.sync_copy(rowsb, out_hbm.at[pl.ds(dstbase + 128, PT - 128)])


@functools.partial(
    pl.kernel,
    mesh=plsc.VectorSubcoreMesh(core_axis_name="c", subcore_axis_name="s"),
    compiler_params=pltpu.CompilerParams(needs_layout_passes=False),
    out_type=[
        jax.ShapeDtypeStruct((ROWS, IN_DIM), jnp.float32),
        jax.ShapeDtypeStruct((BS, 16), jnp.int32),
    ],
    scratch_types=[
        pltpu.VMEM((SEQ,), jnp.int32),
        pltpu.VMEM((128,), jnp.int32),
        pltpu.VMEM((PT - 128,), jnp.int32),
        pltpu.VMEM((128, IN_DIM), jnp.float32),
        pltpu.VMEM((PT - 128, IN_DIM), jnp.float32),
        pltpu.VMEM((16,), jnp.int32),
        pltpu.SemaphoreType.DMA,
        pltpu.SemaphoreType.DMA,
    ],
)
def _sc_gather(text_hbm, feat_hbm, out_hbm, lens_hbm, *scratch):
    _sc_body(text_hbm, feat_hbm, out_hbm, lens_hbm, *scratch)


NBLK = 8
BLK = ROWS // NBLK


def _tc_body(x_hbm, wlt_hbm, w0t_hbm, w1t_hbm, blin_ref, b0_ref, b1_ref,
             g0_ref, be0_ref, lens_ref, out_ref, xbuf, hbuf,
             wlv, w0v, w1v, sems, wsems):
    cw0 = pltpu.make_async_copy(w0t_hbm, w0v, wsems.at[0])
    cw0.start()
    copies = []
    for k in range(NBLK):
        copies.append(pltpu.make_async_copy(
            x_hbm.at[pl.ds(k * BLK, BLK)], xbuf.at[pl.ds(k * BLK, BLK)],
            sems.at[k]))
        copies[-1].start()
    cw1 = pltpu.make_async_copy(w1t_hbm, w1v, wsems.at[1])
    cw1.start()
    cwl = pltpu.make_async_copy(wlt_hbm, wlv, wsems.at[2])
    cwl.start()
    cw0.wait()
    w0t_ref, w1t_ref, wlt_ref = w0v, w1v, wlv

    denom = jnp.float32(BS * KK)
    s1 = jnp.zeros((1, HID), jnp.float32)
    s2 = jnp.zeros((1, HID), jnp.float32)
    half_ok = lax.broadcasted_iota(jnp.int32, (BLK, 1), 0) < (KK - BLK)
    for k in range(NBLK):
        copies[k].wait()
        xk = xbuf[k * BLK:(k + 1) * BLK, :]
        if k % 2 == 1:
            xk = jnp.where(half_ok, xk, 0.0)
        ssq = jnp.sum(xk * xk, axis=1, keepdims=True)
        xn = xk / jnp.maximum(jnp.sqrt(ssq), 1e-6)
        xbuf[k * BLK:(k + 1) * BLK, :] = xn
        h = jnp.dot(xn, w0t_ref[...],
                    preferred_element_type=jnp.float32) + b0_ref[...]
        hm = jnp.where(half_ok, h, 0.0) if k % 2 == 1 else h
        s1 = s1 + jnp.sum(hm, axis=0, keepdims=True)
        s2 = s2 + jnp.sum(hm * hm, axis=0, keepdims=True)
        hbuf[k * BLK:(k + 1) * BLK, :] = h

    mu = s1 / denom
    var = s2 / denom - mu * mu
    rstd = 1.0 / jnp.sqrt(var + 1e-5)

    cw1.wait()
    cwl.wait()
    neg = jnp.float32(-jnp.inf)
    biota = lax.broadcasted_iota(jnp.int32, (BLK, 1), 0)
    for k in range(NBLK):
        b = k // 2
        h = hbuf[k * BLK:(k + 1) * BLK, :]
        hn = (h - mu) * rstd * g0_ref[...] + be0_ref[...]
        r = jnp.maximum(hn, 0.0)
        fused = (jnp.dot(r, w1t_ref[...], preferred_element_type=jnp.float32)
                 + jnp.dot(xbuf[k * BLK:(k + 1) * BLK, :], wlt_ref[...],
                           preferred_element_type=jnp.float32)
                 + b1_ref[...] + blin_ref[...])
        plen = jnp.clip(SEQ - lens_ref[b, 0] - 2, 1, KK)
        pm = biota + (k % 2) * BLK < plen
        bmax = jnp.max(jnp.where(pm, fused, neg), axis=0)
        if k % 2 == 0:
            out_ref[b, :] = bmax
        else:
            out_ref[b, :] = jnp.maximum(out_ref[b, :], bmax)


def _tc_dense(x, wlt, w0t, w1t, blin, b0, b1, g0, be0, lens):
    vspec = pl.BlockSpec(memory_space=pltpu.VMEM)
    return pl.pallas_call(
        _tc_body,
        out_shape=jax.ShapeDtypeStruct((BS, EMB), jnp.float32),
        in_specs=[pl.BlockSpec(memory_space=pl.ANY)] * 4 + [vspec] * 5
        + [pl.BlockSpec(memory_space=pltpu.SMEM)],
        out_specs=vspec,
        scratch_shapes=[
            pltpu.VMEM((ROWS, IN_DIM), jnp.float32),
            pltpu.VMEM((ROWS, HID), jnp.float32),
            pltpu.VMEM((IN_DIM, EMB), jnp.float32),
            pltpu.VMEM((IN_DIM, HID), jnp.float32),
            pltpu.VMEM((HID, EMB), jnp.float32),
            pltpu.SemaphoreType.DMA((NBLK,)),
            pltpu.SemaphoreType.DMA((3,)),
        ],
        compiler_params=pltpu.CompilerParams(
            vmem_limit_bytes=128 * 1024 * 1024),
    )(x, wlt, w0t, w1t, blin, b0, b1, g0, be0, lens)


def kernel(features, text, atten, W_lin, b_lin, W0, b0, g0, be0, W1, b1):
    del atten
    feat_flat = features.reshape(BS * SEQ, IN_DIM)
    text_flat = text.reshape(BS * SEQ).astype(jnp.int32)

    feats_sc, lens = _sc_gather(text_flat, feat_flat)

    out = _tc_dense(
        feats_sc,
        W_lin.T, W0.T, W1.T,
        b_lin.reshape(1, EMB), b0.reshape(1, HID), b1.reshape(1, EMB),
        g0.reshape(1, HID), be0.reshape(1, HID),
        lens,
    )
    return out.astype(jnp.float32)


# speedup vs baseline: 1.1693x; 1.0090x over previous
import functools

import jax
import jax.numpy as jnp
from jax import lax
from jax.experimental import pallas as pl
from jax.experimental.pallas import tpu as pltpu
from jax.experimental.pallas import tpu_sc as plsc

BS, SEQ, IN_DIM, EMB = 4, 4096, 512, 1024
HID = EMB // 2
KK = max(1, int((SEQ - 2) * 0.3))
KPAD = 1280
PT = KPAD // 8
ROWS = BS * KPAD
NCH = SEQ // 16
NDMA = 4
CR = PT // NDMA


def _sc_body(text_hbm, feat_hbm, out_hbm, lens_hbm,
             trow, idxbuf, rowsa, rowsb, nzv, sema, semb):
    c = lax.axis_index("c")
    sub = lax.axis_index("s")
    wid = 16 * c + sub
    b = wid // 8
    s = wid % 8
    lo = s * PT
    dstbase = b * KPAD + lo

    pltpu.sync_copy(text_hbm.at[pl.ds(b * SEQ, SEQ)], trow)

    lane = lax.iota(jnp.int32, 16)

    def abody(i, zt):
        t = trow[pl.ds(i * 16, 16)]
        return zt + plsc.all_reduce_population_count(t == 0)
    z_total = lax.fori_loop(0, NCH, abody, jnp.zeros((16,), jnp.int32))

    @pl.when(s == 0)
    def _():
        nzv[...] = z_total
        pltpu.sync_copy(nzv, lens_hbm.at[b])

    lo_vec = jnp.full((16,), lo, jnp.int32)
    src_base = jnp.full((16,), b * SEQ, jnp.int32) + lane

    def bbody(i, z_run):
        t = trow[pl.ds(i * 16, 16)]
        zb = t == 0
        zc = plsc.cumsum(zb.astype(jnp.int32))
        pos = i * 16 + lane
        rank = jnp.where(zb, z_run + zc,
                         z_total + pos + 1 - z_run - zc) - 1
        slot = rank - lo_vec
        m = (slot >= 0) & (slot < PT)
        srcv = src_base + i * 16
        plsc.store_scatter(idxbuf, [jnp.clip(slot, 0, PT - 1)], srcv, mask=m)
        return z_run + plsc.all_reduce_population_count(zb)

    zt0 = z_total[0]
    chunk_end = jnp.minimum(NCH, (lo + PT + zt0 + 15) // 16)
    z_run1 = lax.fori_loop(0, chunk_end, bbody, jnp.zeros((16,), jnp.int32))

    @pl.when(zt0 > lo)
    def _():
        lax.fori_loop(chunk_end, NCH, bbody, z_run1)

    bufs = (rowsa, rowsb)
    sems = (sema, semb)
    copies = []
    for j in range(NDMA):
        copies.append(pltpu.make_async_copy(
            feat_hbm.at[idxbuf.at[pl.ds(j * CR, CR)]], bufs[j % 2],
            sems[j % 2]))
    copies[0].start()
    for j in range(NDMA):
        if j + 1 < NDMA:
            copies[j + 1].start()
        copies[j].wait()
        pltpu.sync_copy(bufs[j % 2], out_hbm.at[pl.ds(dstbase + j * CR, CR)])


@functools.partial(
    pl.kernel,
    mesh=plsc.VectorSubcoreMesh(core_axis_name="c", subcore_axis_name="s"),
    compiler_params=pltpu.CompilerParams(needs_layout_passes=False),
    out_type=[
        jax.ShapeDtypeStruct((ROWS, IN_DIM), jnp.float32),
        jax.ShapeDtypeStruct((BS, 16), jnp.int32),
    ],
    scratch_types=[
        pltpu.VMEM((SEQ,), jnp.int32),
        pltpu.VMEM((PT,), jnp.int32),
        pltpu.VMEM((CR, IN_DIM), jnp.float32),
        pltpu.VMEM((CR, IN_DIM), jnp.float32),
        pltpu.VMEM((16,), jnp.int32),
        pltpu.SemaphoreType.DMA,
        pltpu.SemaphoreType.DMA,
    ],
)
def _sc_gather(text_hbm, feat_hbm, out_hbm, lens_hbm, *scratch):
    _sc_body(text_hbm, feat_hbm, out_hbm, lens_hbm, *scratch)


NBLK = 8
BLK = ROWS // NBLK


def _tc_body(x_hbm, wlt_hbm, w0t_hbm, w1t_hbm, blin_ref, b0_ref, b1_ref,
             g0_ref, be0_ref, lens_ref, out_ref, xbuf, hbuf,
             wlv, w0v, w1v, sems, wsems):
    cw0 = pltpu.make_async_copy(w0t_hbm, w0v, wsems.at[0])
    cw0.start()
    copies = []
    for k in range(NBLK):
        copies.append(pltpu.make_async_copy(
            x_hbm.at[pl.ds(k * BLK, BLK)], xbuf.at[pl.ds(k * BLK, BLK)],
            sems.at[k]))
        copies[-1].start()
    cw1 = pltpu.make_async_copy(w1t_hbm, w1v, wsems.at[1])
    cw1.start()
    cwl = pltpu.make_async_copy(wlt_hbm, wlv, wsems.at[2])
    cwl.start()
    cw0.wait()
    w0t_ref, w1t_ref, wlt_ref = w0v, w1v, wlv

    denom = jnp.float32(BS * KK)
    s1 = jnp.zeros((1, HID), jnp.float32)
    s2 = jnp.zeros((1, HID), jnp.float32)
    half_ok = lax.broadcasted_iota(jnp.int32, (BLK, 1), 0) < (KK - BLK)
    for k in range(NBLK):
        copies[k].wait()
        xk = xbuf[k * BLK:(k + 1) * BLK, :]
        if k % 2 == 1:
            xk = jnp.where(half_ok, xk, 0.0)
        ssq = jnp.sum(xk * xk, axis=1, keepdims=True)
        xn = xk / jnp.maximum(jnp.sqrt(ssq), 1e-6)
        xbuf[k * BLK:(k + 1) * BLK, :] = xn
        h = jnp.dot(xn, w0t_ref[...],
                    preferred_element_type=jnp.float32) + b0_ref[...]
        hm = jnp.where(half_ok, h, 0.0) if k % 2 == 1 else h
        s1 = s1 + jnp.sum(hm, axis=0, keepdims=True)
        s2 = s2 + jnp.sum(hm * hm, axis=0, keepdims=True)
        hbuf[k * BLK:(k + 1) * BLK, :] = h

    mu = s1 / denom
    var = s2 / denom - mu * mu
    rstd = 1.0 / jnp.sqrt(var + 1e-5)

    cw1.wait()
    cwl.wait()
    neg = jnp.float32(-jnp.inf)
    biota = lax.broadcasted_iota(jnp.int32, (BLK, 1), 0)
    for k in range(NBLK):
        b = k // 2
        h = hbuf[k * BLK:(k + 1) * BLK, :]
        hn = (h - mu) * rstd * g0_ref[...] + be0_ref[...]
        r = jnp.maximum(hn, 0.0)
        fused = (jnp.dot(r, w1t_ref[...], preferred_element_type=jnp.float32)
                 + jnp.dot(xbuf[k * BLK:(k + 1) * BLK, :], wlt_ref[...],
                           preferred_element_type=jnp.float32)
                 + b1_ref[...] + blin_ref[...])
        plen = jnp.clip(SEQ - lens_ref[b, 0] - 2, 1, KK)
        pm = biota + (k % 2) * BLK < plen
        bmax = jnp.max(jnp.where(pm, fused, neg), axis=0)
        if k % 2 == 0:
            out_ref[b, :] = bmax
        else:
            out_ref[b, :] = jnp.maximum(out_ref[b, :], bmax)


def _tc_dense(x, wlt, w0t, w1t, blin, b0, b1, g0, be0, lens):
    vspec = pl.BlockSpec(memory_space=pltpu.VMEM)
    return pl.pallas_call(
        _tc_body,
        out_shape=jax.ShapeDtypeStruct((BS, EMB), jnp.float32),
        in_specs=[pl.BlockSpec(memory_space=pl.ANY)] * 4 + [vspec] * 5
        + [pl.BlockSpec(memory_space=pltpu.SMEM)],
        out_specs=vspec,
        scratch_shapes=[
            pltpu.VMEM((ROWS, IN_DIM), jnp.float32),
            pltpu.VMEM((ROWS, HID), jnp.float32),
            pltpu.VMEM((IN_DIM, EMB), jnp.float32),
            pltpu.VMEM((IN_DIM, HID), jnp.float32),
            pltpu.VMEM((HID, EMB), jnp.float32),
            pltpu.SemaphoreType.DMA((NBLK,)),
            pltpu.SemaphoreType.DMA((3,)),
        ],
        compiler_params=pltpu.CompilerParams(
            vmem_limit_bytes=128 * 1024 * 1024),
    )(x, wlt, w0t, w1t, blin, b0, b1, g0, be0, lens)


def kernel(features, text, atten, W_lin, b_lin, W0, b0, g0, be0, W1, b1):
    del atten
    feat_flat = features.reshape(BS * SEQ, IN_DIM)
    text_flat = text.reshape(BS * SEQ).astype(jnp.int32)

    feats_sc, lens = _sc_gather(text_flat, feat_flat)

    out = _tc_dense(
        feats_sc,
        W_lin.T, W0.T, W1.T,
        b_lin.reshape(1, EMB), b0.reshape(1, HID), b1.reshape(1, EMB),
        g0.reshape(1, HID), be0.reshape(1, HID),
        lens,
    )
    return out.astype(jnp.float32)
